# baseline (speedup 1.0000x reference)
import jax
import jax.numpy as jnp
from jax.experimental import pallas as pl
from jax.experimental.pallas import tpu as pltpu

_EMB = 32
_K = 8192
_CHUNK = 2048
_NCHUNK = _K // _CHUNK
_ROWS = 256
_N = 8192
_NT = _N // _ROWS
_VQ_COMMIT = 0.25


def _vq_body(x_ref, w_ref, rs_ref, q_ref, ind_ref, diff_ref, perp_ref,
             counts_ref, acc_ref):
    i = pl.program_id(0)
    x = x_ref[...]
    w = w_ref[...]
    rs = rs_ref[...]

    cs = jnp.sum(w ** 2, axis=0, keepdims=True)
    mm = jnp.dot(x, w, preferred_element_type=jnp.float32)
    dist = rs - 2.0 * mm + cs

    accv = jnp.full((_ROWS,), jnp.inf, jnp.float32)
    acci = jnp.zeros((_ROWS,), jnp.int32)
    iota_c = jax.lax.broadcasted_iota(jnp.int32, (_ROWS, _CHUNK), 1)
    for c in range(_NCHUNK):
        dch = dist[:, c * _CHUNK:(c + 1) * _CHUNK]
        mck = jnp.min(dch, axis=1, keepdims=True)
        ac = jnp.min(jnp.where(dch == mck, iota_c, _CHUNK),
                     axis=1) + c * _CHUNK
        mc = mck[:, 0]
        beat = mc < accv
        accv = jnp.where(beat, mc.astype(jnp.bfloat16).astype(jnp.float32),
                         accv)
        acci = jnp.where(beat, ac, acci)
    ind = acci

    onehot = (jax.lax.broadcasted_iota(jnp.int32, (_ROWS, _K), 1)
              == ind[:, None]).astype(jnp.float32)
    q = jax.lax.dot_general(onehot, w, (((1,), (1,)), ((), ())),
                            precision=jax.lax.Precision.HIGHEST,
                            preferred_element_type=jnp.float32)

    q_ref[...] = x + (q - x)
    ind_ref[0, 0, :] = ind

    pcounts = jnp.sum(onehot, axis=0)[None, :]
    psq = jnp.sum((q - x) ** 2)

    @pl.when(i == 0)
    def _():
        counts_ref[...] = pcounts
        acc_ref[0, 0] = psq

    @pl.when(i > 0)
    def _():
        counts_ref[...] += pcounts
        acc_ref[0, 0] += psq

    @pl.when(i == _NT - 1)
    def _():
        total = acc_ref[0, 0] / (_N * _EMB)
        diff_ref[...] = jnp.full((1, 1), _VQ_COMMIT * total + total,
                                 jnp.float32)
        avg = counts_ref[...] / _N
        ent = jnp.sum(avg * jnp.log(avg + 1e-10))
        perp_ref[...] = jnp.exp(-jnp.full((1, 1), ent, jnp.float32))


def kernel(input, W):
    x = jnp.swapaxes(input, 1, -1)
    flat = x.reshape(-1, _EMB)
    rowsq = jnp.sum(flat ** 2, axis=1, keepdims=True)

    grid = (_NT,)
    q_flat, ind3, diff, perp = pl.pallas_call(
        _vq_body,
        grid=grid,
        in_specs=[
            pl.BlockSpec((_ROWS, _EMB), lambda i: (i, 0)),
            pl.BlockSpec((_EMB, _K), lambda i: (0, 0)),
            pl.BlockSpec((_ROWS, 1), lambda i: (i, 0)),
        ],
        out_specs=[
            pl.BlockSpec((_ROWS, _EMB), lambda i: (i, 0)),
            pl.BlockSpec((1, 1, _ROWS), lambda i: (i, 0, 0)),
            pl.BlockSpec((1, 1), lambda i: (0, 0)),
            pl.BlockSpec((1, 1), lambda i: (0, 0)),
        ],
        out_shape=[
            jax.ShapeDtypeStruct((_N, _EMB), jnp.float32),
            jax.ShapeDtypeStruct((_NT, 1, _ROWS), jnp.int32),
            jax.ShapeDtypeStruct((1, 1), jnp.float32),
            jax.ShapeDtypeStruct((1, 1), jnp.float32),
        ],
        scratch_shapes=[
            pltpu.VMEM((1, _K), jnp.float32),
            pltpu.SMEM((1, 1), jnp.float32),
        ],
    )(flat, W, rowsq)

    quantize = jnp.swapaxes(q_flat.reshape(x.shape), 1, -1)
    ind_r = ind3.reshape(x.shape[:-1])
    return (quantize, diff.reshape(()), ind_r, perp.reshape(()))


# speedup vs baseline: 2.3334x; 2.3334x over previous
import functools

import jax
import jax.numpy as jnp
from jax import lax
from jax.experimental import pallas as pl
from jax.experimental.pallas import tpu as pltpu
from jax.experimental.pallas import tpu_sc as plsc

_EMB = 32
_K = 8192
_CHUNK = 2048
_NCHUNK = _K // _CHUNK
_ROWS = 256
_N = 8192
_NT = _N // _ROWS
_VQ_COMMIT = 0.25

_NW = 32
_BPW = _N // _NW
_ISEG = 128


def _argmin_body(x_ref, w_ref, rs_ref, ind_ref, diff_ref, acc_ref):
    i = pl.program_id(0)
    x = x_ref[...]
    w = w_ref[...]
    rs = rs_ref[...]

    cs = jnp.sum(w ** 2, axis=0, keepdims=True)
    mm = jnp.dot(x, w, preferred_element_type=jnp.float32)
    dist = rs - 2.0 * mm + cs

    accv = jnp.full((_ROWS,), jnp.inf, jnp.float32)
    selv = jnp.zeros((_ROWS,), jnp.float32)
    acci = jnp.zeros((_ROWS,), jnp.int32)
    iota_c = jax.lax.broadcasted_iota(jnp.int32, (_ROWS, _CHUNK), 1)
    for c in range(_NCHUNK):
        dch = dist[:, c * _CHUNK:(c + 1) * _CHUNK]
        mck = jnp.min(dch, axis=1, keepdims=True)
        ac = jnp.min(jnp.where(dch == mck, iota_c, _CHUNK),
                     axis=1) + c * _CHUNK
        mc = mck[:, 0]
        beat = mc < accv
        accv = jnp.where(beat, mc.astype(jnp.bfloat16).astype(jnp.float32),
                         accv)
        selv = jnp.where(beat, mc, selv)
        acci = jnp.where(beat, ac, acci)

    ind_ref[0, 0, :] = acci
    psq = jnp.sum(selv)

    @pl.when(i == 0)
    def _():
        acc_ref[0, 0] = psq

    @pl.when(i > 0)
    def _():
        acc_ref[0, 0] += psq

    @pl.when(i == _NT - 1)
    def _():
        total = acc_ref[0, 0] / (_N * _EMB)
        diff_ref[...] = jnp.full((1, 1), _VQ_COMMIT * total + total,
                                 jnp.float32)


_sc_mesh = plsc.VectorSubcoreMesh(core_axis_name="c", subcore_axis_name="s")


_ROW128 = 128


@functools.partial(
    pl.kernel, mesh=_sc_mesh,
    out_type=[jax.ShapeDtypeStruct((_N, _ROW128), jnp.float32),
              jax.ShapeDtypeStruct((2, _K), jnp.float32)],
    scratch_types=[pltpu.VMEM((_BPW,), jnp.int32),
                   pltpu.VMEM((_BPW, _ROW128), jnp.float32),
                   pltpu.VMEM((_BPW,), jnp.float32),
                   pltpu.VMEM_SHARED((_K,), jnp.float32),
                   pltpu.SemaphoreType.DMA])
def _sc_gather_hist(table_hbm, idx_hbm, zeros_hbm, ones_hbm, q_hbm, cnt_hbm,
                    idx_v, rows_v, ones_v, shared, sem):
    c = lax.axis_index("c")
    s = lax.axis_index("s")
    wid = s * 2 + c
    base = wid * _BPW
    pltpu.sync_copy(idx_hbm.at[pl.ds(base, _BPW)], idx_v)
    pltpu.sync_copy(ones_hbm.at[pl.ds(0, _BPW)], ones_v)
    for j in range(_BPW // _ISEG):
        pltpu.async_copy(
            table_hbm.at[idx_v.at[pl.ds(j * _ISEG, _ISEG)]],
            rows_v.at[pl.ds(j * _ISEG, _ISEG)], sem).wait()
    pltpu.sync_copy(rows_v, q_hbm.at[pl.ds(base, _BPW)])

    @pl.when(s == 0)
    def _():
        pltpu.sync_copy(zeros_hbm, shared)

    plsc.subcore_barrier()
    for j in range(_BPW // _ISEG):
        pltpu.sync_copy(ones_v.at[pl.ds(j * _ISEG, _ISEG)],
                        shared.at[idx_v.at[pl.ds(j * _ISEG, _ISEG)]],
                        add=True)
    plsc.subcore_barrier()

    @pl.when(s == 0)
    def _():
        pltpu.sync_copy(shared, cnt_hbm.at[c])


def _finish_body(q_ref, x_ref, cnt_ref, quant_ref, perp_ref):
    q = q_ref[:, :_EMB]
    x = x_ref[...]
    quant_ref[...] = x + (q - x)
    counts = cnt_ref[0:1, :] + cnt_ref[1:2, :]
    avg = counts / _N
    ent = jnp.sum(avg * jnp.log(avg + 1e-10))
    perp_ref[...] = jnp.exp(-jnp.full((1, 1), ent, jnp.float32))


def kernel(input, W):
    x = jnp.swapaxes(input, 1, -1)
    flat = x.reshape(-1, _EMB)
    rowsq = jnp.sum(flat ** 2, axis=1, keepdims=True)
    table = jnp.pad(jnp.swapaxes(W, 0, 1),
                    ((0, 0), (0, _ROW128 - _EMB)))

    ind3, diff = pl.pallas_call(
        _argmin_body,
        grid=(_NT,),
        in_specs=[
            pl.BlockSpec((_ROWS, _EMB), lambda i: (i, 0)),
            pl.BlockSpec((_EMB, _K), lambda i: (0, 0)),
            pl.BlockSpec((_ROWS, 1), lambda i: (i, 0)),
        ],
        out_specs=[
            pl.BlockSpec((1, 1, _ROWS), lambda i: (i, 0, 0)),
            pl.BlockSpec((1, 1), lambda i: (0, 0)),
        ],
        out_shape=[
            jax.ShapeDtypeStruct((_NT, 1, _ROWS), jnp.int32),
            jax.ShapeDtypeStruct((1, 1), jnp.float32),
        ],
        scratch_shapes=[
            pltpu.SMEM((1, 1), jnp.float32),
        ],
    )(flat, W, rowsq)

    ind_flat = ind3.reshape(_N)
    zeros = jnp.zeros((_K,), jnp.float32)
    ones = jnp.ones((_BPW,), jnp.float32)
    q_raw, cnt2 = _sc_gather_hist(table, ind_flat, zeros, ones)

    quant_flat, perp = pl.pallas_call(
        _finish_body,
        out_shape=[
            jax.ShapeDtypeStruct((_N, _EMB), jnp.float32),
            jax.ShapeDtypeStruct((1, 1), jnp.float32),
        ],
    )(q_raw, flat, cnt2)

    quantize = jnp.swapaxes(quant_flat.reshape(x.shape), 1, -1)
    ind_r = ind_flat.reshape(x.shape[:-1])
    return (quantize, diff.reshape(()), ind_r, perp.reshape(()))
